# Initial kernel scaffold; baseline (speedup 1.0000x reference)
import jax
import jax.numpy as jnp
from jax.experimental import pallas as pl


def kernel(x, edge_index, batch, Ws, bs, Wh, bh, Wf, bf):
    raise NotImplementedError("write your pallas kernel here")


# speedup vs baseline: 1.5895x; 1.5895x over previous
import functools

import jax
import jax.numpy as jnp
from jax import lax
from jax.experimental import pallas as pl
from jax.experimental.pallas import tpu as pltpu
from jax.experimental.pallas import tpu_sc as plsc

NN = 100000
NE = 1600000
DF = 50
DP = 64
NL = 16
NG = 64

SUB_N = 512
NSUB = 196
NP_ = SUB_N * NSUB
CHUNK = 512
NWORK = 32
SUBS_PER_W = 7

BLK = 1024
NBLK = NP_ // BLK


def _pre_body(x_ref, w_ref, dis_ref, dinv_ref, u_ref, s_ref):
    t = jnp.dot(x_ref[...], w_ref[...], preferred_element_type=jnp.float32)
    u_ref[...] = dis_ref[...] * t
    s_ref[...] = dinv_ref[...] * t


def _mid_body(agg_ref, s_ref, b_ref, w_ref, dis_ref, dinv_ref, u_ref, sn_ref):
    h = jnp.maximum(dis_ref[...] * agg_ref[...] + s_ref[...] + b_ref[...], 0.0)
    t = jnp.dot(h, w_ref[...], preferred_element_type=jnp.float32)
    u_ref[...] = dis_ref[...] * t
    sn_ref[...] = dinv_ref[...] * t


def _fin_body(agg_ref, s_ref, b_ref, dis_ref, batch_ref, wh_ref, bh_ref,
              wf_ref, bf_ref, o_ref, sums_ref, cnts_ref):
    i = pl.program_id(0)

    @pl.when(i == 0)
    def _init():
        sums_ref[...] = jnp.zeros_like(sums_ref)
        cnts_ref[...] = jnp.zeros_like(cnts_ref)

    h = jnp.maximum(dis_ref[...] * agg_ref[...] + s_ref[...] + b_ref[...], 0.0)
    onehot = (batch_ref[...] == lax.broadcasted_iota(jnp.int32, (1, NG), 1)
              ).astype(jnp.float32)
    sums_ref[...] += lax.dot_general(onehot, h, (((0,), (0,)), ((), ())),
                                     preferred_element_type=jnp.float32)
    ones = jnp.ones((BLK, 1), jnp.float32)
    cnts_ref[...] += lax.dot_general(onehot, ones, (((0,), (0,)), ((), ())),
                                     preferred_element_type=jnp.float32)

    @pl.when(i == NBLK - 1)
    def _head():
        pooled = sums_ref[...] / jnp.maximum(cnts_ref[...], 1.0)
        z = jnp.maximum(
            jnp.dot(pooled, wh_ref[...], preferred_element_type=jnp.float32)
            + bh_ref[...], 0.0)
        logits = (jnp.dot(z, wf_ref[...], preferred_element_type=jnp.float32)
                  + bf_ref[...])
        o_ref[...] = 1.0 / (1.0 + jnp.exp(-logits))


_row = pl.BlockSpec((BLK, DP), lambda i: (i, 0))
_col = pl.BlockSpec((BLK, 1), lambda i: (i, 0))
_wspec = pl.BlockSpec((DP, DP), lambda i: (0, 0))
_bspec = pl.BlockSpec((1, DP), lambda i: (0, 0))

_pre = pl.pallas_call(
    _pre_body,
    grid=(NBLK,),
    in_specs=[_row, _wspec, _col, _col],
    out_specs=[_row, _row],
    out_shape=[jax.ShapeDtypeStruct((NP_, DP), jnp.float32)] * 2,
)

_mid = pl.pallas_call(
    _mid_body,
    grid=(NBLK,),
    in_specs=[_row, _row, _bspec, _wspec, _col, _col],
    out_specs=[_row, _row],
    out_shape=[jax.ShapeDtypeStruct((NP_, DP), jnp.float32)] * 2,
)

_fin = pl.pallas_call(
    _fin_body,
    grid=(NBLK,),
    in_specs=[_row, _row, _bspec, _col, _col, _wspec, _bspec,
              pl.BlockSpec((DP, 128), lambda i: (0, 0)),
              pl.BlockSpec((1, 128), lambda i: (0, 0))],
    out_specs=pl.BlockSpec((NG, 128), lambda i: (0, 0)),
    out_shape=jax.ShapeDtypeStruct((NG, 128), jnp.float32),
    scratch_shapes=[pltpu.VMEM((NG, DP), jnp.float32),
                    pltpu.VMEM((NG, 1), jnp.float32)],
)


_sc_mesh = plsc.VectorSubcoreMesh(core_axis_name="c", subcore_axis_name="s")


@functools.partial(
    pl.kernel,
    out_type=jax.ShapeDtypeStruct((NP_, DP), jnp.float32),
    mesh=_sc_mesh,
    compiler_params=pltpu.CompilerParams(use_tc_tiling_on_sc=False),
    scratch_types=[
        pltpu.VMEM((CHUNK,), jnp.int32),
        pltpu.VMEM((CHUNK,), jnp.int32),
        pltpu.VMEM((CHUNK, DP), jnp.float32),
        pltpu.VMEM((SUB_N + 8, DP), jnp.float32),
        pltpu.VMEM((256,), jnp.int32),
        pltpu.SemaphoreType.DMA,
    ],
)
def _spmm(u_hbm, srcs_hbm, dsts_hbm, subs_hbm, agg_hbm,
          idx_v, dst_v, buf_v, acc_v, subs_v, sem):
    wid = lax.axis_index("s") * 2 + lax.axis_index("c")
    pltpu.sync_copy(subs_hbm, subs_v)
    zero16 = jnp.zeros((16,), jnp.float32)

    for kk in range(SUBS_PER_W):
        k = wid + NWORK * kk

        @pl.when(k < NSUB)
        def _sub_block():
            base = k * SUB_N
            sv = subs_v[pl.ds(k, 16)]
            e_lo = sv[0]
            e_hi = sv[1]
            alo = (e_lo // 8) * 8
            nchunks = (e_hi - alo + CHUNK - 1) // CHUNK

            def _zero(r, _):
                for j in range(DP // 16):
                    acc_v[r, pl.ds(16 * j, 16)] = zero16
                return 0

            lax.fori_loop(0, SUB_N + 8, _zero, 0, unroll=4)

            def _chunk(c, _):
                cstart = alo + c * CHUNK
                pltpu.sync_copy(srcs_hbm.at[pl.ds(cstart, CHUNK)], idx_v)
                pltpu.sync_copy(dsts_hbm.at[pl.ds(cstart, CHUNK)], dst_v)
                pltpu.async_copy(u_hbm.at[idx_v], buf_v, sem).wait()

                def _edge16(q, _):
                    dlv = dst_v[pl.ds(q * 16, 16)] - base
                    okv = (dlv >= 0) & (dlv < SUB_N)
                    dlc = jnp.where(okv, dlv, SUB_N)
                    for l in range(16):
                        dl = dlc[l]
                        e = q * 16 + l
                        for j in range(DP // 16):
                            sl = pl.ds(16 * j, 16)
                            plsc.addupdate(acc_v.at[dl, sl], buf_v[e, sl])
                    return 0

                lax.fori_loop(0, CHUNK // 16, _edge16, 0)
                return 0

            lax.fori_loop(0, nchunks, _chunk, 0)
            pltpu.sync_copy(acc_v.at[pl.ds(0, SUB_N)],
                            agg_hbm.at[pl.ds(base, SUB_N)])


def kernel(x, edge_index, batch, Ws, bs, Wh, bh, Wf, bf):
    src = edge_index[0]
    dst = edge_index[1]

    order = jnp.argsort(dst)
    dst_s = dst[order].astype(jnp.int32)
    src_s = src[order].astype(jnp.int32)

    rp = jnp.searchsorted(dst_s, jnp.arange(NN + 1, dtype=jnp.int32),
                          side="left").astype(jnp.int32)
    deg = (rp[1:] - rp[:-1] + 1).astype(jnp.float32)
    dis = lax.rsqrt(deg)
    dinv = 1.0 / deg

    bnds = (jnp.arange(NSUB + 1, dtype=jnp.int32) * SUB_N).astype(jnp.int32)
    subs = jnp.searchsorted(dst_s, bnds, side="left").astype(jnp.int32)
    subs = jnp.concatenate(
        [subs, jnp.full((256 - NSUB - 1,), NE, jnp.int32)])

    src_p = jnp.concatenate([src_s, jnp.zeros((CHUNK,), jnp.int32)])
    dst_p = jnp.concatenate([dst_s, jnp.full((CHUNK,), NP_, jnp.int32)])

    x_p = jnp.zeros((NP_, DP), jnp.float32).at[:NN, :DF].set(x)
    dis_p = jnp.zeros((NP_, 1), jnp.float32).at[:NN, 0].set(dis)
    dinv_p = jnp.zeros((NP_, 1), jnp.float32).at[:NN, 0].set(dinv)
    batch_p = jnp.concatenate(
        [batch.astype(jnp.int32), jnp.full((NP_ - NN,), NG, jnp.int32)]
    ).reshape(NP_, 1)

    Ws_p = jnp.zeros((NL, DP, DP), jnp.float32).at[:, :DF, :DF].set(Ws)
    bs_p = jnp.zeros((NL, 1, DP), jnp.float32).at[:, 0, :DF].set(bs)
    Wh_p = jnp.zeros((DP, DP), jnp.float32).at[:DF, :DF].set(Wh)
    bh_p = jnp.zeros((1, DP), jnp.float32).at[0, :DF].set(bh)
    Wf_p = jnp.zeros((DP, 128), jnp.float32).at[:DF, :1].set(Wf)
    bf_p = jnp.zeros((1, 128), jnp.float32).at[0, :1].set(bf)

    u, s = _pre(x_p, Ws_p[0], dis_p, dinv_p)
    agg = None
    for i in range(NL):
        agg = _spmm(u, src_p, dst_p, subs)
        if i < NL - 1:
            u, s = _mid(agg, s, bs_p[i], Ws_p[i + 1], dis_p, dinv_p)

    out = _fin(agg, s, bs_p[NL - 1], dis_p, batch_p, Wh_p, bh_p, Wf_p, bf_p)
    return out[:, :1]


# speedup vs baseline: 6.1338x; 3.8590x over previous
import functools

import jax
import jax.numpy as jnp
from jax import lax
from jax.experimental import pallas as pl
from jax.experimental.pallas import tpu as pltpu
from jax.experimental.pallas import tpu_sc as plsc

NN = 100000
NE = 1600000
DF = 50
DP = 64
NL = 16
NG = 64

SUB_N = 512
NSUB = 196
NP_ = SUB_N * NSUB
CHUNK = 512
NWORK = 32
SUBS_PER_W = 7

BLK = 1024
NBLK = NP_ // BLK


def _pre_body(x_ref, w_ref, dis_ref, dinv_ref, u_ref, s_ref):
    t = jnp.dot(x_ref[...], w_ref[...], preferred_element_type=jnp.float32)
    u_ref[...] = dis_ref[...] * t
    s_ref[...] = dinv_ref[...] * t


def _mid_body(agg_ref, s_ref, b_ref, w_ref, dis_ref, dinv_ref, u_ref, sn_ref):
    h = jnp.maximum(dis_ref[...] * agg_ref[...] + s_ref[...] + b_ref[...], 0.0)
    t = jnp.dot(h, w_ref[...], preferred_element_type=jnp.float32)
    u_ref[...] = dis_ref[...] * t
    sn_ref[...] = dinv_ref[...] * t


def _fin_body(agg_ref, s_ref, b_ref, dis_ref, batch_ref, wh_ref, bh_ref,
              wf_ref, bf_ref, o_ref, sums_ref, cnts_ref):
    i = pl.program_id(0)

    @pl.when(i == 0)
    def _init():
        sums_ref[...] = jnp.zeros_like(sums_ref)
        cnts_ref[...] = jnp.zeros_like(cnts_ref)

    h = jnp.maximum(dis_ref[...] * agg_ref[...] + s_ref[...] + b_ref[...], 0.0)
    onehot = (batch_ref[...] == lax.broadcasted_iota(jnp.int32, (1, NG), 1)
              ).astype(jnp.float32)
    sums_ref[...] += lax.dot_general(onehot, h, (((0,), (0,)), ((), ())),
                                     preferred_element_type=jnp.float32)
    ones = jnp.ones((BLK, 1), jnp.float32)
    cnts_ref[...] += lax.dot_general(onehot, ones, (((0,), (0,)), ((), ())),
                                     preferred_element_type=jnp.float32)

    @pl.when(i == NBLK - 1)
    def _head():
        pooled = sums_ref[...] / jnp.maximum(cnts_ref[...], 1.0)
        z = jnp.maximum(
            jnp.dot(pooled, wh_ref[...], preferred_element_type=jnp.float32)
            + bh_ref[...], 0.0)
        logits = (jnp.dot(z, wf_ref[...], preferred_element_type=jnp.float32)
                  + bf_ref[...])
        o_ref[...] = 1.0 / (1.0 + jnp.exp(-logits))


_row = pl.BlockSpec((BLK, DP), lambda i: (i, 0))
_col = pl.BlockSpec((BLK, 1), lambda i: (i, 0))
_wspec = pl.BlockSpec((DP, DP), lambda i: (0, 0))
_bspec = pl.BlockSpec((1, DP), lambda i: (0, 0))

_pre = pl.pallas_call(
    _pre_body,
    grid=(NBLK,),
    in_specs=[_row, _wspec, _col, _col],
    out_specs=[_row, _row],
    out_shape=[jax.ShapeDtypeStruct((NP_, DP), jnp.float32)] * 2,
)

_mid = pl.pallas_call(
    _mid_body,
    grid=(NBLK,),
    in_specs=[_row, _row, _bspec, _wspec, _col, _col],
    out_specs=[_row, _row],
    out_shape=[jax.ShapeDtypeStruct((NP_, DP), jnp.float32)] * 2,
)

_fin = pl.pallas_call(
    _fin_body,
    grid=(NBLK,),
    in_specs=[_row, _row, _bspec, _col, _col, _wspec, _bspec,
              pl.BlockSpec((DP, 128), lambda i: (0, 0)),
              pl.BlockSpec((1, 128), lambda i: (0, 0))],
    out_specs=pl.BlockSpec((NG, 128), lambda i: (0, 0)),
    out_shape=jax.ShapeDtypeStruct((NG, 128), jnp.float32),
    scratch_shapes=[pltpu.VMEM((NG, DP), jnp.float32),
                    pltpu.VMEM((NG, 1), jnp.float32)],
)


_sc_mesh = plsc.VectorSubcoreMesh(core_axis_name="c", subcore_axis_name="s")


@functools.partial(
    pl.kernel,
    out_type=jax.ShapeDtypeStruct((NP_, DP), jnp.float32),
    mesh=_sc_mesh,
    compiler_params=pltpu.CompilerParams(use_tc_tiling_on_sc=False),
    scratch_types=[
        pltpu.VMEM((CHUNK,), jnp.int32),
        pltpu.VMEM((CHUNK,), jnp.int32),
        pltpu.VMEM((CHUNK, DP), jnp.float32),
        pltpu.VMEM((SUB_N + 8, DP), jnp.float32),
        pltpu.VMEM((256,), jnp.int32),
        pltpu.SemaphoreType.DMA,
    ],
)
def _spmm(u_hbm, srcs_hbm, dsts_hbm, subs_hbm, agg_hbm,
          idx_v, dst_v, buf_v, acc_v, subs_v, sem):
    wid = lax.axis_index("s") * 2 + lax.axis_index("c")
    pltpu.sync_copy(subs_hbm, subs_v)
    zero16 = jnp.zeros((16,), jnp.float32)

    for kk in range(SUBS_PER_W):
        k = wid + NWORK * kk

        @pl.when(k < NSUB)
        def _sub_block():
            base = k * SUB_N
            sv = subs_v[pl.ds(k, 16)]
            e_lo = sv[0]
            e_hi = sv[1]
            alo = (e_lo // 8) * 8
            nchunks = (e_hi - alo + CHUNK - 1) // CHUNK

            def _zero(r, _):
                for j in range(DP // 16):
                    acc_v[r, pl.ds(16 * j, 16)] = zero16
                return 0

            lax.fori_loop(0, SUB_N + 8, _zero, 0, unroll=4)

            def _chunk(c, _):
                cstart = alo + c * CHUNK
                pltpu.sync_copy(srcs_hbm.at[pl.ds(cstart, CHUNK)], idx_v)
                pltpu.sync_copy(dsts_hbm.at[pl.ds(cstart, CHUNK)], dst_v)
                pltpu.async_copy(u_hbm.at[idx_v], buf_v, sem).wait()

                def _edge16(q, _):
                    dlv = dst_v[pl.ds(q * 16, 16)] - base
                    okv = (dlv >= 0) & (dlv < SUB_N)
                    dlc = jnp.where(okv, dlv, SUB_N)
                    for l in range(16):
                        dl = dlc[l]
                        e = q * 16 + l
                        for j in range(DP // 16):
                            sl = pl.ds(16 * j, 16)
                            plsc.addupdate(acc_v.at[dl, sl], buf_v[e, sl])
                    return 0

                lax.fori_loop(0, CHUNK // 16, _edge16, 0)
                return 0

            lax.fori_loop(0, nchunks, _chunk, 0)
            pltpu.sync_copy(acc_v.at[pl.ds(0, SUB_N)],
                            agg_hbm.at[pl.ds(base, SUB_N)])


def kernel(x, edge_index, batch, Ws, bs, Wh, bh, Wf, bf):
    src = edge_index[0]
    dst = edge_index[1]

    dst_s, src_s = lax.sort((dst.astype(jnp.int32), src.astype(jnp.int32)),
                            num_keys=1, is_stable=False)

    indeg = jnp.zeros((NN,), jnp.int32).at[dst].add(1)
    deg = (indeg + 1).astype(jnp.float32)
    dis = lax.rsqrt(deg)
    dinv = 1.0 / deg

    indeg_p = jnp.concatenate([indeg, jnp.zeros((NP_ - NN,), jnp.int32)])
    per_sub = jnp.sum(indeg_p.reshape(NSUB, SUB_N), axis=1)
    subs = jnp.concatenate(
        [jnp.zeros((1,), jnp.int32), jnp.cumsum(per_sub).astype(jnp.int32),
         jnp.full((256 - NSUB - 1,), NE, jnp.int32)])

    src_p = jnp.concatenate([src_s, jnp.zeros((CHUNK,), jnp.int32)])
    dst_p = jnp.concatenate([dst_s, jnp.full((CHUNK,), NP_, jnp.int32)])

    x_p = jnp.zeros((NP_, DP), jnp.float32).at[:NN, :DF].set(x)
    dis_p = jnp.zeros((NP_, 1), jnp.float32).at[:NN, 0].set(dis)
    dinv_p = jnp.zeros((NP_, 1), jnp.float32).at[:NN, 0].set(dinv)
    batch_p = jnp.concatenate(
        [batch.astype(jnp.int32), jnp.full((NP_ - NN,), NG, jnp.int32)]
    ).reshape(NP_, 1)

    Ws_p = jnp.zeros((NL, DP, DP), jnp.float32).at[:, :DF, :DF].set(Ws)
    bs_p = jnp.zeros((NL, 1, DP), jnp.float32).at[:, 0, :DF].set(bs)
    Wh_p = jnp.zeros((DP, DP), jnp.float32).at[:DF, :DF].set(Wh)
    bh_p = jnp.zeros((1, DP), jnp.float32).at[0, :DF].set(bh)
    Wf_p = jnp.zeros((DP, 128), jnp.float32).at[:DF, :1].set(Wf)
    bf_p = jnp.zeros((1, 128), jnp.float32).at[0, :1].set(bf)

    u, s = _pre(x_p, Ws_p[0], dis_p, dinv_p)
    agg = None
    for i in range(NL):
        agg = _spmm(u, src_p, dst_p, subs)
        if i < NL - 1:
            u, s = _mid(agg, s, bs_p[i], Ws_p[i + 1], dis_p, dinv_p)

    out = _fin(agg, s, bs_p[NL - 1], dis_p, batch_p, Wh_p, bh_p, Wf_p, bf_p)
    return out[:, :1]


# speedup vs baseline: 7.0888x; 1.1557x over previous
import functools

import jax
import jax.numpy as jnp
from jax import lax
from jax.experimental import pallas as pl
from jax.experimental.pallas import tpu as pltpu
from jax.experimental.pallas import tpu_sc as plsc

NN = 100000
NE = 1600000
DF = 50
DP = 64
NL = 16
NG = 64

SUB_N = 512
NSUB = 196
NP_ = SUB_N * NSUB
CHUNK = 512
NWORK = 32
SUBS_PER_W = 7

BLK = 1024
NBLK = NP_ // BLK


def _pre_body(x_ref, w_ref, dis_ref, dinv_ref, u_ref, s_ref):
    t = jnp.dot(x_ref[...], w_ref[...], preferred_element_type=jnp.float32)
    u_ref[...] = dis_ref[...] * t
    s_ref[...] = dinv_ref[...] * t


def _mid_body(agg_ref, s_ref, b_ref, w_ref, dis_ref, dinv_ref, u_ref, sn_ref):
    h = jnp.maximum(dis_ref[...] * agg_ref[...] + s_ref[...] + b_ref[...], 0.0)
    t = jnp.dot(h, w_ref[...], preferred_element_type=jnp.float32)
    u_ref[...] = dis_ref[...] * t
    sn_ref[...] = dinv_ref[...] * t


def _fin_body(agg_ref, s_ref, b_ref, dis_ref, batch_ref, wh_ref, bh_ref,
              wf_ref, bf_ref, o_ref, sums_ref, cnts_ref):
    i = pl.program_id(0)

    @pl.when(i == 0)
    def _init():
        sums_ref[...] = jnp.zeros_like(sums_ref)
        cnts_ref[...] = jnp.zeros_like(cnts_ref)

    h = jnp.maximum(dis_ref[...] * agg_ref[...] + s_ref[...] + b_ref[...], 0.0)
    onehot = (batch_ref[...] == lax.broadcasted_iota(jnp.int32, (1, NG), 1)
              ).astype(jnp.float32)
    sums_ref[...] += lax.dot_general(onehot, h, (((0,), (0,)), ((), ())),
                                     preferred_element_type=jnp.float32)
    ones = jnp.ones((BLK, 1), jnp.float32)
    cnts_ref[...] += lax.dot_general(onehot, ones, (((0,), (0,)), ((), ())),
                                     preferred_element_type=jnp.float32)

    @pl.when(i == NBLK - 1)
    def _head():
        pooled = sums_ref[...] / jnp.maximum(cnts_ref[...], 1.0)
        z = jnp.maximum(
            jnp.dot(pooled, wh_ref[...], preferred_element_type=jnp.float32)
            + bh_ref[...], 0.0)
        logits = (jnp.dot(z, wf_ref[...], preferred_element_type=jnp.float32)
                  + bf_ref[...])
        o_ref[...] = 1.0 / (1.0 + jnp.exp(-logits))


_row = pl.BlockSpec((BLK, DP), lambda i: (i, 0))
_col = pl.BlockSpec((BLK, 1), lambda i: (i, 0))
_wspec = pl.BlockSpec((DP, DP), lambda i: (0, 0))
_bspec = pl.BlockSpec((1, DP), lambda i: (0, 0))

_pre = pl.pallas_call(
    _pre_body,
    grid=(NBLK,),
    in_specs=[_row, _wspec, _col, _col],
    out_specs=[_row, _row],
    out_shape=[jax.ShapeDtypeStruct((NP_, DP), jnp.float32)] * 2,
)

_mid = pl.pallas_call(
    _mid_body,
    grid=(NBLK,),
    in_specs=[_row, _row, _bspec, _wspec, _col, _col],
    out_specs=[_row, _row],
    out_shape=[jax.ShapeDtypeStruct((NP_, DP), jnp.float32)] * 2,
)

_fin = pl.pallas_call(
    _fin_body,
    grid=(NBLK,),
    in_specs=[_row, _row, _bspec, _col, _col, _wspec, _bspec,
              pl.BlockSpec((DP, 128), lambda i: (0, 0)),
              pl.BlockSpec((1, 128), lambda i: (0, 0))],
    out_specs=pl.BlockSpec((NG, 128), lambda i: (0, 0)),
    out_shape=jax.ShapeDtypeStruct((NG, 128), jnp.float32),
    scratch_shapes=[pltpu.VMEM((NG, DP), jnp.float32),
                    pltpu.VMEM((NG, 1), jnp.float32)],
)


_sc_mesh = plsc.VectorSubcoreMesh(core_axis_name="c", subcore_axis_name="s")


@functools.partial(
    pl.kernel,
    out_type=jax.ShapeDtypeStruct((NP_, DP), jnp.float32),
    mesh=_sc_mesh,
    compiler_params=pltpu.CompilerParams(use_tc_tiling_on_sc=False),
    scratch_types=[
        pltpu.VMEM((CHUNK,), jnp.int32),
        pltpu.VMEM((CHUNK,), jnp.int32),
        pltpu.VMEM((CHUNK,), jnp.int32),
        pltpu.VMEM((CHUNK,), jnp.int32),
        pltpu.VMEM((CHUNK, DP), jnp.float32),
        pltpu.VMEM((CHUNK, DP), jnp.float32),
        pltpu.VMEM((SUB_N + 8, DP), jnp.float32),
        pltpu.VMEM((256,), jnp.int32),
        pltpu.SemaphoreType.DMA,
        pltpu.SemaphoreType.DMA,
    ],
)
def _spmm(u_hbm, srcs_hbm, dsts_hbm, subs_hbm, agg_hbm,
          idx0_v, idx1_v, dst0_v, dst1_v, buf0_v, buf1_v,
          acc_v, subs_v, sem0, sem1):
    idx_b = (idx0_v, idx1_v)
    dst_b = (dst0_v, dst1_v)
    buf_b = (buf0_v, buf1_v)
    sem_b = (sem0, sem1)
    wid = lax.axis_index("s") * 2 + lax.axis_index("c")
    pltpu.sync_copy(subs_hbm, subs_v)
    zero16 = jnp.zeros((16,), jnp.float32)

    for kk in range(SUBS_PER_W):
        k = wid + NWORK * kk

        @pl.when(k < NSUB)
        def _sub_block():
            base = k * SUB_N
            sv = subs_v[pl.ds(k, 16)]
            e_lo = sv[0]
            e_hi = sv[1]
            alo = (e_lo // 8) * 8
            nchunks = (e_hi - alo + CHUNK - 1) // CHUNK

            def _zero(r, _):
                for j in range(DP // 16):
                    acc_v[r, pl.ds(16 * j, 16)] = zero16
                return 0

            lax.fori_loop(0, SUB_N + 8, _zero, 0, unroll=4)

            def _issue(c, s):
                cstart = alo + c * CHUNK
                pltpu.sync_copy(srcs_hbm.at[pl.ds(cstart, CHUNK)], idx_b[s])
                pltpu.sync_copy(dsts_hbm.at[pl.ds(cstart, CHUNK)], dst_b[s])
                pltpu.async_copy(u_hbm.at[idx_b[s]], buf_b[s], sem_b[s])

            def _process(s):
                dv, bv = dst_b[s], buf_b[s]

                def _edge16(q, _):
                    dlv = dv[pl.ds(q * 16, 16)] - base
                    okv = (dlv >= 0) & (dlv < SUB_N)
                    dlc = jnp.where(okv, dlv, SUB_N)
                    for l in range(16):
                        dl = dlc[l]
                        e = q * 16 + l
                        for j in range(DP // 16):
                            sl = pl.ds(16 * j, 16)
                            plsc.addupdate(acc_v.at[dl, sl], bv[e, sl])
                    return 0

                lax.fori_loop(0, CHUNK // 16, _edge16, 0)

            def _wait(s):
                pltpu.make_async_copy(u_hbm.at[idx_b[s]], buf_b[s],
                                      sem_b[s]).wait()

            @pl.when(nchunks > 0)
            def _prime():
                _issue(0, 0)

            def _pair(p, _):
                c = 2 * p

                @pl.when(c + 1 < nchunks)
                def _():
                    _issue(c + 1, 1)

                _wait(0)
                _process(0)

                @pl.when(c + 2 < nchunks)
                def _():
                    _issue(c + 2, 0)

                @pl.when(c + 1 < nchunks)
                def _():
                    _wait(1)
                    _process(1)

                return 0

            lax.fori_loop(0, (nchunks + 1) // 2, _pair, 0)
            pltpu.sync_copy(acc_v.at[pl.ds(0, SUB_N)],
                            agg_hbm.at[pl.ds(base, SUB_N)])


def kernel(x, edge_index, batch, Ws, bs, Wh, bh, Wf, bf):
    src = edge_index[0]
    dst = edge_index[1]

    dst_s, src_s = lax.sort((dst.astype(jnp.int32), src.astype(jnp.int32)),
                            num_keys=1, is_stable=False)

    indeg = jnp.zeros((NN,), jnp.int32).at[dst].add(1)
    deg = (indeg + 1).astype(jnp.float32)
    dis = lax.rsqrt(deg)
    dinv = 1.0 / deg

    indeg_p = jnp.concatenate([indeg, jnp.zeros((NP_ - NN,), jnp.int32)])
    per_sub = jnp.sum(indeg_p.reshape(NSUB, SUB_N), axis=1)
    subs = jnp.concatenate(
        [jnp.zeros((1,), jnp.int32), jnp.cumsum(per_sub).astype(jnp.int32),
         jnp.full((256 - NSUB - 1,), NE, jnp.int32)])

    src_p = jnp.concatenate([src_s, jnp.zeros((CHUNK,), jnp.int32)])
    dst_p = jnp.concatenate([dst_s, jnp.full((CHUNK,), NP_, jnp.int32)])

    x_p = jnp.zeros((NP_, DP), jnp.float32).at[:NN, :DF].set(x)
    dis_p = jnp.zeros((NP_, 1), jnp.float32).at[:NN, 0].set(dis)
    dinv_p = jnp.zeros((NP_, 1), jnp.float32).at[:NN, 0].set(dinv)
    batch_p = jnp.concatenate(
        [batch.astype(jnp.int32), jnp.full((NP_ - NN,), NG, jnp.int32)]
    ).reshape(NP_, 1)

    Ws_p = jnp.zeros((NL, DP, DP), jnp.float32).at[:, :DF, :DF].set(Ws)
    bs_p = jnp.zeros((NL, 1, DP), jnp.float32).at[:, 0, :DF].set(bs)
    Wh_p = jnp.zeros((DP, DP), jnp.float32).at[:DF, :DF].set(Wh)
    bh_p = jnp.zeros((1, DP), jnp.float32).at[0, :DF].set(bh)
    Wf_p = jnp.zeros((DP, 128), jnp.float32).at[:DF, :1].set(Wf)
    bf_p = jnp.zeros((1, 128), jnp.float32).at[0, :1].set(bf)

    u, s = _pre(x_p, Ws_p[0], dis_p, dinv_p)
    agg = None
    for i in range(NL):
        agg = _spmm(u, src_p, dst_p, subs)
        if i < NL - 1:
            u, s = _mid(agg, s, bs_p[i], Ws_p[i + 1], dis_p, dinv_p)

    out = _fin(agg, s, bs_p[NL - 1], dis_p, batch_p, Wh_p, bh_p, Wf_p, bf_p)
    return out[:, :1]


# speedup vs baseline: 14.1897x; 2.0017x over previous
import functools

import jax
import jax.numpy as jnp
from jax import lax
from jax.experimental import pallas as pl
from jax.experimental.pallas import tpu as pltpu
from jax.experimental.pallas import tpu_sc as plsc

NN = 100000
NE = 1600000
DF = 50
DP = 64
NL = 16
NG = 64

SUB_N = 512
NSUB = 196
NP_ = SUB_N * NSUB
CHUNK = 512
NWORK = 32
SUBS_PER_W = 7

BLK = 1024
NBLK = NP_ // BLK


def _pre_body(x_ref, w_ref, dis_ref, dinv_ref, u_ref, s_ref):
    t = jnp.dot(x_ref[...], w_ref[...], preferred_element_type=jnp.float32)
    u_ref[...] = dis_ref[...] * t
    s_ref[...] = dinv_ref[...] * t


def _mid_body(agg_ref, s_ref, b_ref, w_ref, dis_ref, dinv_ref, u_ref, sn_ref):
    h = jnp.maximum(dis_ref[...] * agg_ref[...] + s_ref[...] + b_ref[...], 0.0)
    t = jnp.dot(h, w_ref[...], preferred_element_type=jnp.float32)
    u_ref[...] = dis_ref[...] * t
    sn_ref[...] = dinv_ref[...] * t


def _fin_body(agg_ref, s_ref, b_ref, dis_ref, batch_ref, wh_ref, bh_ref,
              wf_ref, bf_ref, o_ref, sums_ref, cnts_ref):
    i = pl.program_id(0)

    @pl.when(i == 0)
    def _init():
        sums_ref[...] = jnp.zeros_like(sums_ref)
        cnts_ref[...] = jnp.zeros_like(cnts_ref)

    h = jnp.maximum(dis_ref[...] * agg_ref[...] + s_ref[...] + b_ref[...], 0.0)
    onehot = (batch_ref[...] == lax.broadcasted_iota(jnp.int32, (1, NG), 1)
              ).astype(jnp.float32)
    sums_ref[...] += lax.dot_general(onehot, h, (((0,), (0,)), ((), ())),
                                     preferred_element_type=jnp.float32)
    ones = jnp.ones((BLK, 1), jnp.float32)
    cnts_ref[...] += lax.dot_general(onehot, ones, (((0,), (0,)), ((), ())),
                                     preferred_element_type=jnp.float32)

    @pl.when(i == NBLK - 1)
    def _head():
        pooled = sums_ref[...] / jnp.maximum(cnts_ref[...], 1.0)
        z = jnp.maximum(
            jnp.dot(pooled, wh_ref[...], preferred_element_type=jnp.float32)
            + bh_ref[...], 0.0)
        logits = (jnp.dot(z, wf_ref[...], preferred_element_type=jnp.float32)
                  + bf_ref[...])
        o_ref[...] = 1.0 / (1.0 + jnp.exp(-logits))


_row = pl.BlockSpec((BLK, DP), lambda i: (i, 0))
_col = pl.BlockSpec((BLK, 1), lambda i: (i, 0))
_wspec = pl.BlockSpec((DP, DP), lambda i: (0, 0))
_bspec = pl.BlockSpec((1, DP), lambda i: (0, 0))

_pre = pl.pallas_call(
    _pre_body,
    grid=(NBLK,),
    in_specs=[_row, _wspec, _col, _col],
    out_specs=[_row, _row],
    out_shape=[jax.ShapeDtypeStruct((NP_, DP), jnp.float32)] * 2,
)

_mid = pl.pallas_call(
    _mid_body,
    grid=(NBLK,),
    in_specs=[_row, _row, _bspec, _wspec, _col, _col],
    out_specs=[_row, _row],
    out_shape=[jax.ShapeDtypeStruct((NP_, DP), jnp.float32)] * 2,
)

_fin = pl.pallas_call(
    _fin_body,
    grid=(NBLK,),
    in_specs=[_row, _row, _bspec, _col, _col, _wspec, _bspec,
              pl.BlockSpec((DP, 128), lambda i: (0, 0)),
              pl.BlockSpec((1, 128), lambda i: (0, 0))],
    out_specs=pl.BlockSpec((NG, 128), lambda i: (0, 0)),
    out_shape=jax.ShapeDtypeStruct((NG, 128), jnp.float32),
    scratch_shapes=[pltpu.VMEM((NG, DP), jnp.float32),
                    pltpu.VMEM((NG, 1), jnp.float32)],
)


_sc_mesh = plsc.VectorSubcoreMesh(core_axis_name="c", subcore_axis_name="s")


SUB2 = NP_ // 14
ACC2 = SUB2 + 16
ZROWS = ACC2 // 16
OROWS = SUB2 // 16


@functools.partial(
    pl.kernel,
    out_type=jax.ShapeDtypeStruct((NP_, DP), jnp.float32),
    mesh=_sc_mesh,
    compiler_params=pltpu.CompilerParams(use_tc_tiling_on_sc=False),
    scratch_types=[
        pltpu.VMEM((CHUNK,), jnp.int32),
        pltpu.VMEM((CHUNK,), jnp.int32),
        pltpu.VMEM((CHUNK,), jnp.int32),
        pltpu.VMEM((CHUNK,), jnp.int32),
        pltpu.VMEM((4, 128), jnp.int32),
        pltpu.VMEM((4, 128), jnp.int32),
        pltpu.VMEM((CHUNK, DP), jnp.float32),
        pltpu.VMEM((CHUNK, DP), jnp.float32),
        pltpu.VMEM((CHUNK, DP), jnp.float32),
        pltpu.VMEM_SHARED((ACC2, DP), jnp.float32),
        pltpu.VMEM((256,), jnp.int32),
        pltpu.SemaphoreType.DMA,
        pltpu.SemaphoreType.DMA,
    ],
)
def _spmm(u_hbm, srcs_hbm, dsts_hbm, subs_hbm, agg_hbm,
          idx0_v, idx1_v, dst0_v, dst1_v, dlc0_v, dlc1_v, buf0_v, buf1_v,
          zbuf_v, acc_sh, subs_v, sem0, sem1):
    idx_b = (idx0_v, idx1_v)
    dst_b = (dst0_v, dst1_v)
    dlc_b = (dlc0_v, dlc1_v)
    buf_b = (buf0_v, buf1_v)
    sem_b = (sem0, sem1)
    core = lax.axis_index("c")
    tid = lax.axis_index("s")
    pltpu.sync_copy(subs_hbm, subs_v)
    zero16 = jnp.zeros((16,), jnp.float32)

    def _zero(r, _):
        for j in range(DP // 16):
            zbuf_v[r, pl.ds(16 * j, 16)] = zero16
        return 0

    lax.fori_loop(0, CHUNK, _zero, 0, unroll=4)

    for kk in range(7):
        k = 2 * kk + core
        base = k * SUB2
        sv = subs_v[pl.ds(k, 16)]
        e_lo = sv[0]
        e_hi = sv[1]
        alo = (e_lo // 8) * 8
        nchunks = (e_hi - alo + CHUNK - 1) // CHUNK
        nmine = jnp.maximum(nchunks - tid + 15, 0) // 16

        z0 = tid * ZROWS
        pltpu.sync_copy(zbuf_v.at[pl.ds(0, ZROWS)],
                        acc_sh.at[pl.ds(z0, ZROWS)])
        plsc.subcore_barrier()

        def _issue(m, s):
            cstart = alo + (m * 16 + tid) * CHUNK
            pltpu.sync_copy(srcs_hbm.at[pl.ds(cstart, CHUNK)], idx_b[s])
            pltpu.sync_copy(dsts_hbm.at[pl.ds(cstart, CHUNK)], dst_b[s])
            pltpu.async_copy(u_hbm.at[idx_b[s]], buf_b[s], sem_b[s])

        def _process(s):
            dv, lv, bv = dst_b[s], dlc_b[s], buf_b[s]
            for q in range(CHUNK // 16):
                dlv = dv[pl.ds(q * 16, 16)] - base
                okv = (dlv >= 0) & (dlv < SUB2)
                dlc = jnp.where(okv, dlv, SUB2 + tid)
                lv[q // 8, pl.ds(16 * (q % 8), 16)] = dlc
            for i in range(4):
                pltpu.sync_copy(bv.at[pl.ds(128 * i, 128)],
                                acc_sh.at[lv.at[i]], add=True)

        def _wait(s):
            pltpu.make_async_copy(u_hbm.at[idx_b[s]], buf_b[s],
                                  sem_b[s]).wait()

        @pl.when(nmine > 0)
        def _prime():
            _issue(0, 0)

        def _pair(p, _):
            m = 2 * p

            @pl.when(m + 1 < nmine)
            def _():
                _issue(m + 1, 1)

            _wait(0)
            _process(0)

            @pl.when(m + 2 < nmine)
            def _():
                _issue(m + 2, 0)

            @pl.when(m + 1 < nmine)
            def _():
                _wait(1)
                _process(1)

            return 0

        lax.fori_loop(0, (nmine + 1) // 2, _pair, 0)
        plsc.subcore_barrier()

        o0 = tid * OROWS
        pltpu.sync_copy(acc_sh.at[pl.ds(o0, OROWS)],
                        zbuf_v.at[pl.ds(0, OROWS)])
        pltpu.sync_copy(zbuf_v.at[pl.ds(0, OROWS)],
                        agg_hbm.at[pl.ds(base + o0, OROWS)])
        lax.fori_loop(0, OROWS, _zero, 0, unroll=4)
        plsc.subcore_barrier()


def kernel(x, edge_index, batch, Ws, bs, Wh, bh, Wf, bf):
    src = edge_index[0]
    dst = edge_index[1]

    dst_s, src_s = lax.sort((dst.astype(jnp.int32), src.astype(jnp.int32)),
                            num_keys=1, is_stable=False)

    indeg = jnp.zeros((NN,), jnp.int32).at[dst].add(1)
    deg = (indeg + 1).astype(jnp.float32)
    dis = lax.rsqrt(deg)
    dinv = 1.0 / deg

    indeg_p = jnp.concatenate([indeg, jnp.zeros((NP_ - NN,), jnp.int32)])
    per_sub = jnp.sum(indeg_p.reshape(14, SUB2), axis=1)
    subs = jnp.concatenate(
        [jnp.zeros((1,), jnp.int32), jnp.cumsum(per_sub).astype(jnp.int32),
         jnp.full((256 - 15,), NE, jnp.int32)])

    src_p = jnp.concatenate([src_s, jnp.zeros((CHUNK,), jnp.int32)])
    dst_p = jnp.concatenate([dst_s, jnp.full((CHUNK,), NP_, jnp.int32)])

    x_p = jnp.zeros((NP_, DP), jnp.float32).at[:NN, :DF].set(x)
    dis_p = jnp.zeros((NP_, 1), jnp.float32).at[:NN, 0].set(dis)
    dinv_p = jnp.zeros((NP_, 1), jnp.float32).at[:NN, 0].set(dinv)
    batch_p = jnp.concatenate(
        [batch.astype(jnp.int32), jnp.full((NP_ - NN,), NG, jnp.int32)]
    ).reshape(NP_, 1)

    Ws_p = jnp.zeros((NL, DP, DP), jnp.float32).at[:, :DF, :DF].set(Ws)
    bs_p = jnp.zeros((NL, 1, DP), jnp.float32).at[:, 0, :DF].set(bs)
    Wh_p = jnp.zeros((DP, DP), jnp.float32).at[:DF, :DF].set(Wh)
    bh_p = jnp.zeros((1, DP), jnp.float32).at[0, :DF].set(bh)
    Wf_p = jnp.zeros((DP, 128), jnp.float32).at[:DF, :1].set(Wf)
    bf_p = jnp.zeros((1, 128), jnp.float32).at[0, :1].set(bf)

    u, s = _pre(x_p, Ws_p[0], dis_p, dinv_p)
    agg = None
    for i in range(NL):
        agg = _spmm(u, src_p, dst_p, subs)
        if i < NL - 1:
            u, s = _mid(agg, s, bs_p[i], Ws_p[i + 1], dis_p, dinv_p)

    out = _fin(agg, s, bs_p[NL - 1], dis_p, batch_p, Wh_p, bh_p, Wf_p, bf_p)
    return out[:, :1]
